# baseline (speedup 1.0000x reference)
import functools

import jax
import jax.numpy as jnp
from jax import lax
from jax.experimental import pallas as pl
from jax.experimental.pallas import tpu as pltpu
from jax.experimental.pallas import tpu_sc as plsc

_LANES = 128
_K = 2
_CHUNK = _K * _LANES
_PADW = 128


@functools.lru_cache(maxsize=None)
def _make_gather(seq: int, batch: int):
    info = plsc.get_sparse_core_info()
    nc, ns = info.num_cores, info.num_subcores
    nw = nc * ns
    bs_per_w = batch // nw
    rows_per_w = bs_per_w * seq
    n_chunks = rows_per_w // _CHUNK
    n_pairs = n_chunks // 2
    assert batch % (nw * _LANES) == 0 or bs_per_w == _LANES
    assert rows_per_w % _CHUNK == 0 and n_chunks % 2 == 0

    mesh = plsc.VectorSubcoreMesh(core_axis_name="c", subcore_axis_name="s")

    @functools.partial(
        pl.kernel,
        mesh=mesh,
        out_type=jax.ShapeDtypeStruct((batch * seq, _PADW), jnp.float32),
        scratch_types=[
            pltpu.VMEM((seq, _LANES), jnp.int32),
            pltpu.VMEM((rows_per_w,), jnp.int32),
            pltpu.VMEM((_CHUNK, _PADW), jnp.float32),
            pltpu.VMEM((_CHUNK, _PADW), jnp.float32),
            pltpu.SemaphoreType.DMA,
            pltpu.SemaphoreType.DMA,
        ],
        compiler_params=pltpu.CompilerParams(
            needs_layout_passes=False, skip_device_barrier=True
        ),
    )
    def k(table_hbm, idx_hbm, out_hbm, idx_s, idx_v, rows0, rows1, sem0, sem1):
        wid = lax.axis_index("s") * nc + lax.axis_index("c")
        out0 = wid * rows_per_w

        pltpu.sync_copy(idx_hbm.at[:, pl.ds(wid * _LANES, _LANES)], idx_s)

        cvecs = [
            (jnp.arange(16, dtype=jnp.int32) + 16 * bg) * seq for bg in range(8)
        ]

        def trans_body(s, carry):
            sv = jnp.full((16,), s, dtype=jnp.int32)
            for bg in range(8):
                v = idx_s[s, pl.ds(16 * bg, 16)]
                plsc.store_scatter(idx_v, [cvecs[bg] + sv], v)
            return carry

        lax.fori_loop(0, seq, trans_body, 0)

        def fire(g, rows_v, sem):
            for j in range(_K):
                pltpu.async_copy(
                    table_hbm.at[idx_v.at[pl.ds(g * _CHUNK + j * _LANES, _LANES)]],
                    rows_v.at[pl.ds(j * _LANES, _LANES)],
                    sem,
                )

        def drain_wait(sem):
            for j in range(_K):
                pltpu.make_async_copy(
                    table_hbm.at[idx_v.at[pl.ds(j * _LANES, _LANES)]],
                    rows0.at[pl.ds(j * _LANES, _LANES)],
                    sem,
                ).wait()

        def store(g, rows_v):
            pltpu.sync_copy(
                rows_v,
                out_hbm.at[pl.ds(out0 + g * _CHUNK, _CHUNK)],
            )

        fire(0, rows0, sem0)

        def body(p, carry):
            g = 2 * p
            fire(g + 1, rows1, sem1)
            drain_wait(sem0)
            store(g, rows0)
            fire(g + 2, rows0, sem0)
            drain_wait(sem1)
            store(g + 1, rows1)
            return carry

        lax.fori_loop(0, n_pairs - 1, body, 0)

        g_last = n_chunks - 2
        fire(g_last + 1, rows1, sem1)
        drain_wait(sem0)
        store(g_last, rows0)
        drain_wait(sem1)
        store(g_last + 1, rows1)

    return k


def kernel(inputs, embed_table):
    b, s = inputs.shape
    v, d = embed_table.shape
    idx_t = inputs.T.astype(jnp.int32)
    tab_p = jnp.zeros((v, _PADW), jnp.float32).at[:, :d].set(embed_table)
    out = _make_gather(s, b)(tab_p, idx_t)
    return out[:, :d].reshape(b, s, d)


# speedup vs baseline: 1.3138x; 1.3138x over previous
import functools

import jax
import jax.numpy as jnp
from jax import lax
from jax.experimental import pallas as pl
from jax.experimental.pallas import tpu as pltpu
from jax.experimental.pallas import tpu_sc as plsc

_LANES = 128
_PADW = 128


@functools.lru_cache(maxsize=None)
def _make_gather(seq: int, batch: int):
    info = plsc.get_sparse_core_info()
    nc, ns = info.num_cores, info.num_subcores
    nw = nc * ns
    bs_per_w = batch // nw
    rows_per_w = bs_per_w * seq
    n_chunks = rows_per_w // _LANES
    assert bs_per_w == _LANES and rows_per_w % _LANES == 0 and n_chunks % 4 == 0

    mesh = plsc.VectorSubcoreMesh(core_axis_name="c", subcore_axis_name="s")

    @functools.partial(
        pl.kernel,
        mesh=mesh,
        out_type=jax.ShapeDtypeStruct((batch * seq, _PADW), jnp.float32),
        scratch_types=[
            pltpu.VMEM((seq, _LANES), jnp.int32),
            pltpu.VMEM((rows_per_w,), jnp.int32),
            pltpu.VMEM((_LANES, _PADW), jnp.float32),
            pltpu.VMEM((_LANES, _PADW), jnp.float32),
            pltpu.VMEM((_LANES, _PADW), jnp.float32),
            pltpu.VMEM((_LANES, _PADW), jnp.float32),
            pltpu.SemaphoreType.DMA,
            pltpu.SemaphoreType.DMA,
            pltpu.SemaphoreType.DMA,
            pltpu.SemaphoreType.DMA,
        ],
        compiler_params=pltpu.CompilerParams(
            needs_layout_passes=False, skip_device_barrier=True
        ),
    )
    def k(table_hbm, idx_hbm, out_hbm, idx_s, idx_v,
          b0, b1, b2, b3, sm0, sm1, sm2, sm3):
        wid = lax.axis_index("s") * nc + lax.axis_index("c")
        out0 = wid * rows_per_w

        pltpu.sync_copy(idx_hbm.at[:, pl.ds(wid * _LANES, _LANES)], idx_s)

        cvecs = [
            (jnp.arange(16, dtype=jnp.int32) + 16 * bg) * seq for bg in range(8)
        ]

        def trans_body(s, carry):
            sv = jnp.full((16,), s, dtype=jnp.int32)
            for bg in range(8):
                v = idx_s[s, pl.ds(16 * bg, 16)]
                plsc.store_scatter(idx_v, [cvecs[bg] + sv], v)
            return carry

        lax.fori_loop(0, seq, trans_body, 0)

        bufs = (b0, sm0), (b1, sm1), (b2, sm2), (b3, sm3)

        def fire(g, r):
            buf, sem = bufs[r]
            pltpu.async_copy(
                table_hbm.at[idx_v.at[pl.ds(g * _LANES, _LANES)]], buf, sem
            )

        def wait_store(g, r):
            buf, sem = bufs[r]
            pltpu.make_async_copy(
                table_hbm.at[idx_v.at[pl.ds(0, _LANES)]], buf, sem
            ).wait()
            pltpu.sync_copy(buf, out_hbm.at[pl.ds(out0 + g * _LANES, _LANES)])

        fire(0, 0)
        fire(1, 1)

        def body(p, carry):
            g = 4 * p
            fire(g + 2, 2)
            wait_store(g, 0)
            fire(g + 3, 3)
            wait_store(g + 1, 1)
            fire(g + 4, 0)
            wait_store(g + 2, 2)
            fire(g + 5, 1)
            wait_store(g + 3, 3)
            return carry

        lax.fori_loop(0, n_chunks // 4 - 1, body, 0)

        g = n_chunks - 4
        fire(g + 2, 2)
        wait_store(g, 0)
        fire(g + 3, 3)
        wait_store(g + 1, 1)
        wait_store(g + 2, 2)
        wait_store(g + 3, 3)

    return k


def kernel(inputs, embed_table):
    b, s = inputs.shape
    v, d = embed_table.shape
    idx_t = inputs.T.astype(jnp.int32)
    tab_p = jnp.pad(embed_table, ((0, 0), (0, _PADW - d)))
    out = _make_gather(s, b)(tab_p, idx_t)
    return out[:, :d].reshape(b, s, d)
